# baseline (speedup 1.0000x reference)
import functools

import jax
import jax.numpy as jnp
from jax import lax
from jax.experimental import pallas as pl
from jax.experimental.pallas import tpu as pltpu
from jax.experimental.pallas import tpu_sc as plsc

LANES = 16
NUM_CORES = 2
NUM_SUBCORES = 16
NW = NUM_CORES * NUM_SUBCORES


def kernel(x, token_table, pos_table):
    batch, seq = x.shape
    dim = token_table.shape[1]
    n_local = batch // NW

    mesh = plsc.VectorSubcoreMesh(core_axis_name="c", subcore_axis_name="s")

    @functools.partial(
        pl.kernel,
        mesh=mesh,
        out_type=jax.ShapeDtypeStruct((batch * seq, dim), jnp.float32),
        scratch_types=[
            pltpu.VMEM((seq,), jnp.int32),
            pltpu.VMEM((seq, dim), jnp.float32),
            pltpu.VMEM((seq, dim), jnp.float32),
            pltpu.SemaphoreType.DMA,
        ],
        compiler_params=pltpu.CompilerParams(use_tc_tiling_on_sc=False),
    )
    def emb(x_hbm, tok_hbm, pos_hbm, out_hbm, idx_v, rows_v, pos_v, sem):
        wid = lax.axis_index("s") * NUM_CORES + lax.axis_index("c")
        base = wid * n_local
        pltpu.sync_copy(pos_hbm.at[pl.ds(0, seq)], pos_v)

        @pl.loop(0, n_local)
        def _(j):
            pltpu.sync_copy(x_hbm.at[base + j], idx_v)
            c0 = pltpu.async_copy(
                tok_hbm.at[idx_v.at[pl.ds(0, 128)]], rows_v.at[pl.ds(0, 128)], sem
            )
            c1 = pltpu.async_copy(
                tok_hbm.at[idx_v.at[pl.ds(128, seq - 128)]],
                rows_v.at[pl.ds(128, seq - 128)],
                sem,
            )
            c0.wait()
            c1.wait()

            @pl.loop(0, seq)
            def _(r):
                for c in range(dim // LANES):
                    sl = (r, pl.ds(c * LANES, LANES))
                    rows_v[sl] = rows_v[sl] + pos_v[sl]

            pltpu.sync_copy(rows_v, out_hbm.at[pl.ds((base + j) * seq, seq)])

    out = emb(x.astype(jnp.int32), token_table, pos_table)
    return out.reshape(batch, seq, dim)


# speedup vs baseline: 1.2071x; 1.2071x over previous
import functools

import jax
import jax.numpy as jnp
from jax import lax
from jax.experimental import pallas as pl
from jax.experimental.pallas import tpu as pltpu
from jax.experimental.pallas import tpu_sc as plsc

LANES = 16
NUM_CORES = 2
NUM_SUBCORES = 16
NW = NUM_CORES * NUM_SUBCORES
NBUF = 4


def kernel(x, token_table, pos_table):
    batch, seq = x.shape
    dim = token_table.shape[1]
    n_local = batch // NW

    mesh = plsc.VectorSubcoreMesh(core_axis_name="c", subcore_axis_name="s")

    @functools.partial(
        pl.kernel,
        mesh=mesh,
        out_type=jax.ShapeDtypeStruct((batch * seq, dim), jnp.float32),
        scratch_types=[
            pltpu.VMEM((n_local, seq), jnp.int32),
            pltpu.VMEM((seq, dim), jnp.float32),
        ]
        + [pltpu.VMEM((seq, dim), jnp.float32) for _ in range(NBUF)]
        + [pltpu.SemaphoreType.DMA for _ in range(2 * NBUF)],
        compiler_params=pltpu.CompilerParams(use_tc_tiling_on_sc=False),
    )
    def emb(x_hbm, tok_hbm, pos_hbm, out_hbm, idx_all, pos_v, *bufs_and_sems):
        bufs = bufs_and_sems[:NBUF]
        gsems = bufs_and_sems[NBUF : 2 * NBUF]
        osems = bufs_and_sems[2 * NBUF : 3 * NBUF]

        wid = lax.axis_index("s") * NUM_CORES + lax.axis_index("c")
        base = wid * n_local
        pltpu.sync_copy(x_hbm.at[pl.ds(base, n_local)], idx_all)
        pltpu.sync_copy(pos_hbm.at[pl.ds(0, seq)], pos_v)

        def issue_gather(j, buf, sem):
            pltpu.async_copy(
                tok_hbm.at[idx_all.at[j, pl.ds(0, 128)]], buf.at[pl.ds(0, 128)], sem
            )
            pltpu.async_copy(
                tok_hbm.at[idx_all.at[j, pl.ds(128, seq - 128)]],
                buf.at[pl.ds(128, seq - 128)],
                sem,
            )

        def drain_gather(buf, sem):
            pltpu.make_async_copy(tok_hbm.at[pl.ds(0, seq)], buf, sem).wait()

        def drain_out(buf, sem):
            pltpu.make_async_copy(buf, out_hbm.at[pl.ds(0, seq)], sem).wait()

        for b in range(NBUF):
            issue_gather(b, bufs[b], gsems[b])

        @pl.loop(0, n_local, step=NBUF)
        def _(g):
            for b in range(NBUF):
                j = g + b
                sb = (b - 2) % NBUF

                @pl.when(jnp.logical_and(j >= 2, j + 2 < n_local))
                def _():
                    drain_out(bufs[sb], osems[sb])
                    issue_gather(j + 2, bufs[sb], gsems[sb])

                drain_gather(bufs[b], gsems[b])

                @pl.loop(0, seq, step=2)
                def _(r):
                    for rr in range(2):
                        for c in range(dim // LANES):
                            sl = (r + rr, pl.ds(c * LANES, LANES))
                            plsc.addupdate(bufs[b].at[sl], pos_v[sl])

                pltpu.async_copy(
                    bufs[b], out_hbm.at[pl.ds((base + j) * seq, seq)], osems[b]
                )

        for b in range(NBUF):
            drain_out(bufs[b], osems[b])

    out = emb(x.astype(jnp.int32), token_table, pos_table)
    return out.reshape(batch, seq, dim)


# speedup vs baseline: 1.2088x; 1.0015x over previous
import functools

import jax
import jax.numpy as jnp
from jax import lax
from jax.experimental import pallas as pl
from jax.experimental.pallas import tpu as pltpu
from jax.experimental.pallas import tpu_sc as plsc

LANES = 16
NUM_CORES = 2
NUM_SUBCORES = 16
NW = NUM_CORES * NUM_SUBCORES
NBUF = 8
LOOKAHEAD = 6


def kernel(x, token_table, pos_table):
    batch, seq = x.shape
    dim = token_table.shape[1]
    n_local = batch // NW

    mesh = plsc.VectorSubcoreMesh(core_axis_name="c", subcore_axis_name="s")

    @functools.partial(
        pl.kernel,
        mesh=mesh,
        out_type=jax.ShapeDtypeStruct((batch, seq, dim), jnp.float32),
        scratch_types=[
            pltpu.VMEM((seq, n_local), jnp.int32),
            pltpu.VMEM((seq, dim), jnp.float32),
        ]
        + [pltpu.VMEM((n_local, dim), jnp.float32) for _ in range(NBUF)]
        + [pltpu.SemaphoreType.DMA for _ in range(2 * NBUF)],
        compiler_params=pltpu.CompilerParams(use_tc_tiling_on_sc=False),
    )
    def emb(xt_hbm, tok_hbm, pos_hbm, out_hbm, idx_t, pos_v, *bufs_and_sems):
        bufs = bufs_and_sems[:NBUF]
        gsems = bufs_and_sems[NBUF : 2 * NBUF]
        osems = bufs_and_sems[2 * NBUF : 3 * NBUF]

        wid = lax.axis_index("s") * NUM_CORES + lax.axis_index("c")
        base = wid * n_local
        pltpu.sync_copy(xt_hbm.at[:, pl.ds(base, n_local)], idx_t)
        pltpu.sync_copy(pos_hbm.at[pl.ds(0, seq)], pos_v)

        def issue_gather(s, b):
            pltpu.async_copy(tok_hbm.at[idx_t.at[s]], bufs[b], gsems[b])

        def wait_gather(b):
            pltpu.make_async_copy(
                tok_hbm.at[pl.ds(0, n_local)], bufs[b], gsems[b]
            ).wait()

        def issue_out(s, b):
            pltpu.async_copy(bufs[b], out_hbm.at[pl.ds(base, n_local), s], osems[b])

        def wait_out(b):
            pltpu.make_async_copy(
                bufs[b], out_hbm.at[pl.ds(base, n_local), 0], osems[b]
            ).wait()

        for b in range(NBUF):
            issue_gather(b, b)

        @pl.loop(0, seq, step=NBUF)
        def _(g):
            for b in range(NBUF):
                s = g + b
                fb = (b + LOOKAHEAD) % NBUF

                @pl.when(
                    jnp.logical_and(s >= NBUF - LOOKAHEAD, s + LOOKAHEAD < seq)
                )
                def _():
                    wait_out(fb)
                    issue_gather(s + LOOKAHEAD, fb)

                wait_gather(b)

                pv = [pos_v[s, pl.ds(c * LANES, LANES)] for c in range(dim // LANES)]

                @pl.loop(0, n_local, step=4)
                def _(j):
                    for jj in range(4):
                        for c in range(dim // LANES):
                            plsc.addupdate(
                                bufs[b].at[j + jj, pl.ds(c * LANES, LANES)], pv[c]
                            )

                issue_out(s, b)

        for b in range(NBUF):
            wait_out(b)

    return emb(x.T, token_table, pos_table)


# speedup vs baseline: 1.2092x; 1.0003x over previous
import functools

import jax
import jax.numpy as jnp
from jax import lax
from jax.experimental import pallas as pl
from jax.experimental.pallas import tpu as pltpu
from jax.experimental.pallas import tpu_sc as plsc

LANES = 16
NUM_CORES = 2
NUM_SUBCORES = 16
NW = NUM_CORES * NUM_SUBCORES
NBUF = 4


def kernel(x, token_table, pos_table):
    batch, seq = x.shape
    dim = token_table.shape[1]
    n_local = batch // NW

    mesh = plsc.VectorSubcoreMesh(core_axis_name="c", subcore_axis_name="s")

    @functools.partial(
        pl.kernel,
        mesh=mesh,
        out_type=jax.ShapeDtypeStruct((batch * seq, dim), jnp.float32),
        scratch_types=[
            pltpu.VMEM((n_local, seq), jnp.int32),
            pltpu.VMEM((seq, dim), jnp.float32),
        ]
        + [pltpu.VMEM((seq, dim), jnp.float32) for _ in range(NBUF)]
        + [pltpu.SemaphoreType.DMA for _ in range(2 * NBUF)],
        compiler_params=pltpu.CompilerParams(use_tc_tiling_on_sc=False),
    )
    def emb(x_hbm, tok_hbm, pos_hbm, out_hbm, idx_all, pos_v, *bufs_and_sems):
        bufs = bufs_and_sems[:NBUF]
        gsems = bufs_and_sems[NBUF : 2 * NBUF]
        osems = bufs_and_sems[2 * NBUF : 3 * NBUF]

        wid = lax.axis_index("s") * NUM_CORES + lax.axis_index("c")
        base = wid * n_local
        pltpu.sync_copy(x_hbm.at[pl.ds(base, n_local)], idx_all)
        pltpu.sync_copy(pos_hbm.at[pl.ds(0, seq)], pos_v)

        def issue_gather(j, buf, sem):
            pltpu.async_copy(
                tok_hbm.at[idx_all.at[j, pl.ds(0, 128)]], buf.at[pl.ds(0, 128)], sem
            )
            pltpu.async_copy(
                tok_hbm.at[idx_all.at[j, pl.ds(128, seq - 128)]],
                buf.at[pl.ds(128, seq - 128)],
                sem,
            )

        def drain_gather(buf, sem):
            pltpu.make_async_copy(tok_hbm.at[pl.ds(0, seq)], buf, sem).wait()

        def drain_out(buf, sem):
            pltpu.make_async_copy(buf, out_hbm.at[pl.ds(0, seq)], sem).wait()

        for b in range(NBUF):
            issue_gather(b, bufs[b], gsems[b])

        @pl.loop(0, n_local, step=NBUF)
        def _(g):
            for b in range(NBUF):
                j = g + b
                sb = (b - 2) % NBUF

                @pl.when(jnp.logical_and(j >= 2, j + 2 < n_local))
                def _():
                    drain_out(bufs[sb], osems[sb])
                    issue_gather(j + 2, bufs[sb], gsems[sb])

                drain_gather(bufs[b], gsems[b])

                @pl.loop(0, seq, step=2)
                def _(r):
                    for rr in range(2):
                        for c in range(dim // LANES):
                            sl = (r + rr, pl.ds(c * LANES, LANES))
                            plsc.addupdate(bufs[b].at[sl], pos_v[sl])

                pltpu.async_copy(
                    bufs[b], out_hbm.at[pl.ds((base + j) * seq, seq)], osems[b]
                )

        for b in range(NBUF):
            drain_out(bufs[b], osems[b])

    out = emb(x.astype(jnp.int32), token_table, pos_table)
    return out.reshape(batch, seq, dim)


# speedup vs baseline: 1.2095x; 1.0002x over previous
import functools

import jax
import jax.numpy as jnp
from jax import lax
from jax.experimental import pallas as pl
from jax.experimental.pallas import tpu as pltpu
from jax.experimental.pallas import tpu_sc as plsc

LANES = 16
NUM_CORES = 2
NUM_SUBCORES = 16
NW = NUM_CORES * NUM_SUBCORES
NBUF = 4


def kernel(x, token_table, pos_table):
    batch, seq = x.shape
    dim = token_table.shape[1]
    n_local = batch // NW

    mesh = plsc.VectorSubcoreMesh(core_axis_name="c", subcore_axis_name="s")

    @functools.partial(
        pl.kernel,
        mesh=mesh,
        out_type=jax.ShapeDtypeStruct((batch * seq, dim), jnp.float32),
        scratch_types=[
            pltpu.VMEM((n_local, seq), jnp.int32),
            pltpu.VMEM((seq, dim), jnp.float32),
        ]
        + [pltpu.VMEM((seq, dim), jnp.float32) for _ in range(NBUF)]
        + [pltpu.SemaphoreType.DMA for _ in range(2 * NBUF)],
        compiler_params=pltpu.CompilerParams(use_tc_tiling_on_sc=False),
    )
    def emb(x_hbm, tok_hbm, pos_hbm, out_hbm, idx_all, pos_v, *bufs_and_sems):
        bufs = bufs_and_sems[:NBUF]
        gsems = bufs_and_sems[NBUF : 2 * NBUF]
        osems = bufs_and_sems[2 * NBUF : 3 * NBUF]

        wid = lax.axis_index("s") * NUM_CORES + lax.axis_index("c")
        base = wid * n_local
        pltpu.sync_copy(x_hbm.at[pl.ds(base, n_local)], idx_all)
        pltpu.sync_copy(pos_hbm.at[pl.ds(0, seq)], pos_v)

        def issue_gather(j, buf, sem):
            pltpu.async_copy(
                tok_hbm.at[idx_all.at[j, pl.ds(0, 128)]], buf.at[pl.ds(0, 128)], sem
            )
            pltpu.async_copy(
                tok_hbm.at[idx_all.at[j, pl.ds(128, seq - 128)]],
                buf.at[pl.ds(128, seq - 128)],
                sem,
            )

        def drain_gather(buf, sem):
            pltpu.make_async_copy(tok_hbm.at[pl.ds(0, seq)], buf, sem).wait()

        def drain_out(buf, sem):
            pltpu.make_async_copy(buf, out_hbm.at[pl.ds(0, seq)], sem).wait()

        for b in range(NBUF):
            issue_gather(b, bufs[b], gsems[b])

        @pl.loop(0, n_local, step=NBUF)
        def _(g):
            for b in range(NBUF):
                j = g + b
                sb = (b - 2) % NBUF

                @pl.when(jnp.logical_and(j >= 2, j + 2 < n_local))
                def _():
                    drain_out(bufs[sb], osems[sb])
                    issue_gather(j + 2, bufs[sb], gsems[sb])

                drain_gather(bufs[b], gsems[b])

                @pl.loop(0, seq, step=8)
                def _(r):
                    for rr in range(8):
                        for c in range(dim // LANES):
                            sl = (r + rr, pl.ds(c * LANES, LANES))
                            plsc.addupdate(bufs[b].at[sl], pos_v[sl])

                pltpu.async_copy(
                    bufs[b], out_hbm.at[pl.ds((base + j) * seq, seq)], osems[b]
                )

        for b in range(NBUF):
            drain_out(bufs[b], osems[b])

    out = emb(x.astype(jnp.int32), token_table, pos_table)
    return out.reshape(batch, seq, dim)
